# Initial kernel scaffold; baseline (speedup 1.0000x reference)
import jax
import jax.numpy as jnp
from jax.experimental import pallas as pl


def kernel(x, edge_index, W, b):
    raise NotImplementedError("write your pallas kernel here")


# speedup vs baseline: 14.2048x; 14.2048x over previous
import jax
import jax.numpy as jnp
from jax import lax
from jax.experimental import pallas as pl
from jax.experimental.pallas import tpu as pltpu
from jax.experimental.pallas import tpu_sc as plsc

N_NODES = 10000
D_FEAT = 128
N_EDGES = 320000
K_HOPS = 10

NC = 2
NS = 16
DH = D_FEAT // NC
NPADN = 10240
NPT = NPADN // NS
SUBN = 160
NSUB = NPT // SUBN
EPT = N_EDGES // NS
CHUNK = 128
NCHUNK = 160
EPT_PAD = NCHUNK * CHUNK
TRASH = N_NODES


def _sc_body(xs_hbm, rc_hbm, out_hbm,
             g_sh, a_sh,
             idx4, dinv_v, htile, buf2):
    c = lax.axis_index("c")
    s = lax.axis_index("s")
    nbase = s * NPT

    def run(sem_i, sem_g, sem_s):
        def idx_start(j, m):
            pltpu.async_copy(rc_hbm.at[s, j], idx4.at[m], sem_i.at[m])

        def idx_wait(j, m):
            pltpu.make_async_copy(rc_hbm.at[s, j], idx4.at[m],
                                  sem_i.at[m]).wait()

        def gat_start(m, p):
            pltpu.async_copy(g_sh.at[idx4.at[m, 0]], buf2.at[p],
                             sem_g.at[p])

        def gat_wait(m, p):
            pltpu.make_async_copy(g_sh.at[idx4.at[m, 0]], buf2.at[p],
                                  sem_g.at[p]).wait()

        def sca_start(m, p):
            pltpu.async_copy(buf2.at[p], a_sh.at[idx4.at[m, 1]],
                             sem_s.at[p], add=True)

        def sca_wait(m, p):
            pltpu.make_async_copy(buf2.at[p], a_sh.at[idx4.at[m, 1]],
                                  sem_s.at[p]).wait()

        def _zero(n, _):
            for q in range(DH // 16):
                htile[n, pl.ds(q * 16, 16)] = jnp.zeros((16,), jnp.float32)
            return _
        lax.fori_loop(0, SUBN, _zero, None)

        def _zslice(r, _):
            pltpu.sync_copy(htile, a_sh.at[pl.ds(nbase + r * SUBN, SUBN)])
            return _
        lax.fori_loop(0, NSUB, _zslice, None)

        def _ones(n, _):
            for q in range(DH // 16):
                buf2[0, n, pl.ds(q * 16, 16)] = jnp.ones((16,), jnp.float32)
            return _
        lax.fori_loop(0, CHUNK, _ones, None)
        plsc.subcore_barrier()

        idx_start(0, 0)
        idx_start(1, 1)

        def _deg_outer(jo, _):
            for u in range(4):
                j = jo * 4 + u
                m = u
                idx_wait(j, m)
                pltpu.async_copy(buf2.at[0], a_sh.at[idx4.at[m, 1]],
                                 sem_s.at[u % 2], add=True)

                @pl.when(j >= 2)
                def _():
                    pltpu.make_async_copy(
                        buf2.at[0], a_sh.at[idx4.at[(m + 2) % 4, 1]],
                        sem_s.at[u % 2]).wait()

                @pl.when(j + 2 < NCHUNK)
                def _():
                    idx_start(j + 2, (m + 2) % 4)
            return _
        lax.fori_loop(0, NCHUNK // 4, _deg_outer, None)
        pltpu.make_async_copy(buf2.at[0], a_sh.at[idx4.at[2, 1]],
                              sem_s.at[0]).wait()
        pltpu.make_async_copy(buf2.at[0], a_sh.at[idx4.at[3, 1]],
                              sem_s.at[1]).wait()
        plsc.subcore_barrier()

        def _dinv_sub(r, _):
            pltpu.sync_copy(a_sh.at[pl.ds(nbase + r * SUBN, SUBN)], htile)

            def _rsqrt(i, _):
                d = htile[i, pl.ds(0, 16)] + 1.0
                ss = 0.5 * (d + 1.0)
                for _it in range(12):
                    ss = 0.5 * (ss + d / ss)
                dinv_v[r * SUBN + i, :] = 1.0 / ss
                return _
            lax.fori_loop(0, SUBN, _rsqrt, None)
            return _
        lax.fori_loop(0, NSUB, _dinv_sub, None)

        def _scale_sub(r):
            def _srow(i, _):
                dv = dinv_v[r * SUBN + i, :]
                for q in range(DH // 16):
                    sl = pl.ds(q * 16, 16)
                    htile[i, sl] = htile[i, sl] * dv
                return _
            lax.fori_loop(0, SUBN, _srow, None)

        def _hop0_sub(r, _):
            nsl = pl.ds(nbase + r * SUBN, SUBN)
            pltpu.sync_copy(xs_hbm.at[c, nsl], htile)
            pltpu.sync_copy(htile, out_hbm.at[c, 0, nsl])
            _scale_sub(r)
            pltpu.sync_copy(htile, g_sh.at[nsl])
            pltpu.sync_copy(htile, a_sh.at[nsl])
            return _
        lax.fori_loop(0, NSUB, _hop0_sub, None)
        plsc.subcore_barrier()

        def _hop(k, _):
            idx_start(0, 0)
            idx_start(1, 1)
            idx_start(2, 2)
            idx_wait(0, 0)
            gat_start(0, 0)

            def _edge_outer(jo, _):
                for u in range(4):
                    j4 = jo * 4 + u
                    m = u
                    p = u % 2

                    @pl.when(j4 + 1 < NCHUNK)
                    def _():
                        idx_wait(j4 + 1, (m + 1) % 4)

                        @pl.when(j4 >= 1)
                        def _():
                            sca_wait((m + 3) % 4, (p + 1) % 2)
                        gat_start((m + 1) % 4, (p + 1) % 2)

                    @pl.when(j4 + 3 < NCHUNK)
                    def _():
                        idx_start(j4 + 3, (m + 3) % 4)
                    gat_wait(m, p)
                    sca_start(m, p)
                return _
            lax.fori_loop(0, NCHUNK // 4, _edge_outer, None)
            sca_wait(2, 0)
            sca_wait(3, 1)
            plsc.subcore_barrier()

            def _scale_pass(r, _):
                nsl = pl.ds(nbase + r * SUBN, SUBN)
                pltpu.sync_copy(a_sh.at[nsl], htile)
                _scale_sub(r)
                pltpu.sync_copy(htile, out_hbm.at[c, k, nsl])
                _scale_sub(r)
                pltpu.sync_copy(htile, g_sh.at[nsl])
                pltpu.sync_copy(htile, a_sh.at[nsl])
                return _
            lax.fori_loop(0, NSUB, _scale_pass, None)
            plsc.subcore_barrier()
            return _
        lax.fori_loop(1, K_HOPS + 1, _hop, None)

    pl.run_scoped(
        run,
        sem_i=pltpu.SemaphoreType.DMA((4,)),
        sem_g=pltpu.SemaphoreType.DMA((2,)),
        sem_s=pltpu.SemaphoreType.DMA((2,)),
    )


@jax.jit
def _sc_hops(xsplit, rc_p):
    mesh = plsc.VectorSubcoreMesh(
        core_axis_name="c", subcore_axis_name="s",
        num_cores=NC, num_subcores=NS)
    return pl.kernel(
        _sc_body,
        out_type=jax.ShapeDtypeStruct((NC, K_HOPS + 1, NPADN, DH),
                                      jnp.float32),
        mesh=mesh,
        compiler_params=pltpu.CompilerParams(use_tc_tiling_on_sc=False),
        scratch_types=[
            pltpu.VMEM_SHARED((NPADN, DH), jnp.float32),
            pltpu.VMEM_SHARED((NPADN, DH), jnp.float32),
            pltpu.VMEM((4, 2, CHUNK), jnp.int32),
            pltpu.VMEM((NPT, 16), jnp.float32),
            pltpu.VMEM((SUBN, DH), jnp.float32),
            pltpu.VMEM((2, CHUNK, DH), jnp.float32),
        ],
    )(xsplit, rc_p)


def _readout_body(hs_ref, w_ref, b_ref, out_ref):
    hb = hs_ref[...]
    h = jnp.concatenate([hb[0], hb[1]], axis=-1)
    kk, bn, d = h.shape
    logits = jax.lax.dot_general(
        h.reshape(kk * bn, d), w_ref[...],
        (((1,), (0,)), ((), ())),
        preferred_element_type=jnp.float32)
    sig = jax.nn.sigmoid(logits + b_ref[0, 0]).reshape(kk, bn, 1)
    out_ref[...] = jnp.sum(sig * h, axis=0)


BN = 1024


@jax.jit
def _readout(hs, w, b2):
    grid = (NPADN // BN,)
    return pl.pallas_call(
        _readout_body,
        grid=grid,
        in_specs=[
            pl.BlockSpec((NC, K_HOPS + 1, BN, DH), lambda i: (0, 0, i, 0)),
            pl.BlockSpec((D_FEAT, 1), lambda i: (0, 0)),
            pl.BlockSpec((1, 1), lambda i: (0, 0)),
        ],
        out_specs=pl.BlockSpec((BN, D_FEAT), lambda i: (i, 0)),
        out_shape=jax.ShapeDtypeStruct((NPADN, D_FEAT), jnp.float32),
    )(hs, w, b2)


def kernel(x, edge_index, W, b):
    ei = edge_index.astype(jnp.int32)
    row, col = ei[0], ei[1]
    pad = EPT_PAD * NS - N_EDGES
    row_p = jnp.concatenate(
        [row, jnp.zeros((pad,), jnp.int32)]).reshape(NS, NCHUNK, CHUNK)
    col_p = jnp.concatenate(
        [col, jnp.full((pad,), TRASH, jnp.int32)]).reshape(NS, NCHUNK, CHUNK)
    rc_p = jnp.stack([row_p, col_p], axis=2)
    xp = jnp.pad(x, ((0, NPADN - N_NODES), (0, 0)))
    xsplit = xp.reshape(NPADN, NC, DH).transpose(1, 0, 2)
    hs = _sc_hops(xsplit, rc_p)
    return _readout(hs, W, b.reshape(1, 1))[:N_NODES]


# speedup vs baseline: 14.2432x; 1.0027x over previous
import jax
import jax.numpy as jnp
from jax import lax
from jax.experimental import pallas as pl
from jax.experimental.pallas import tpu as pltpu
from jax.experimental.pallas import tpu_sc as plsc

N_NODES = 10000
D_FEAT = 128
N_EDGES = 320000
K_HOPS = 10

NC = 2
NS = 16
DH = D_FEAT // NC
NPADN = 10240
NPT = NPADN // NS
SUBN = 160
NSUB = NPT // SUBN
EPT = N_EDGES // NS
CHUNK = 128
NCHUNK = 168
EPT_PAD = NCHUNK * CHUNK
TRASH = N_NODES


def _sc_body(xs_hbm, rc_hbm, out_hbm,
             g_sh, a_sh,
             idxr, dinv_v, htile, bufr):
    c = lax.axis_index("c")
    s = lax.axis_index("s")
    nbase = s * NPT

    def run(sem_i, sem_g, sem_s):
        def idx_start(j, m):
            pltpu.async_copy(rc_hbm.at[s, j], idxr.at[m], sem_i.at[m])

        def idx_wait(j, m):
            pltpu.make_async_copy(rc_hbm.at[s, j], idxr.at[m],
                                  sem_i.at[m]).wait()

        def gat_start(m, p):
            pltpu.async_copy(g_sh.at[idxr.at[m, 0]], bufr.at[p],
                             sem_g.at[p])

        def gat_wait(m, p):
            pltpu.make_async_copy(g_sh.at[idxr.at[m, 0]], bufr.at[p],
                                  sem_g.at[p]).wait()

        def sca_start(m, p):
            pltpu.async_copy(bufr.at[p], a_sh.at[idxr.at[m, 1]],
                             sem_s.at[p], add=True)

        def sca_wait(m, p):
            pltpu.make_async_copy(bufr.at[p], a_sh.at[idxr.at[m, 1]],
                                  sem_s.at[p]).wait()

        def _zero(n, _):
            for q in range(DH // 16):
                htile[n, pl.ds(q * 16, 16)] = jnp.zeros((16,), jnp.float32)
            return _
        lax.fori_loop(0, SUBN, _zero, None)

        def _zslice(r, _):
            pltpu.sync_copy(htile, a_sh.at[pl.ds(nbase + r * SUBN, SUBN)])
            return _
        lax.fori_loop(0, NSUB, _zslice, None)

        def _ones(n, _):
            for q in range(DH // 16):
                bufr[0, n, pl.ds(q * 16, 16)] = jnp.ones((16,), jnp.float32)
            return _
        lax.fori_loop(0, CHUNK, _ones, None)
        plsc.subcore_barrier()

        def deg_sca_start(m, e):
            pltpu.async_copy(bufr.at[0], a_sh.at[idxr.at[m, 1]],
                             sem_s.at[e], add=True)

        def deg_sca_wait(m, e):
            pltpu.make_async_copy(bufr.at[0], a_sh.at[idxr.at[m, 1]],
                                  sem_s.at[e]).wait()

        for jj in range(4):
            idx_start(jj, jj)

        def _deg_outer(jo, _):
            for u in range(6):
                j = jo * 6 + u
                m = u
                e = u % 3
                idx_wait(j, m)
                deg_sca_start(m, e)

                @pl.when(j >= 2)
                def _():
                    deg_sca_wait((m + 4) % 6, (e + 1) % 3)

                @pl.when(j + 4 < NCHUNK)
                def _():
                    idx_start(j + 4, (m + 4) % 6)
            return _
        lax.fori_loop(0, NCHUNK // 6, _deg_outer, None)
        deg_sca_wait((NCHUNK - 2) % 6, (NCHUNK - 2) % 3)
        deg_sca_wait((NCHUNK - 1) % 6, (NCHUNK - 1) % 3)
        plsc.subcore_barrier()

        def _dinv_sub(r, _):
            pltpu.sync_copy(a_sh.at[pl.ds(nbase + r * SUBN, SUBN)], htile)

            def _rsqrt(i, _):
                d = htile[i, pl.ds(0, 16)] + 1.0
                ss = 0.5 * (d + 1.0)
                for _it in range(12):
                    ss = 0.5 * (ss + d / ss)
                dinv_v[r * SUBN + i, :] = 1.0 / ss
                return _
            lax.fori_loop(0, SUBN, _rsqrt, None)
            return _
        lax.fori_loop(0, NSUB, _dinv_sub, None)

        def _scale_sub(r):
            def _srow(i, _):
                dv = dinv_v[r * SUBN + i, :]
                for q in range(DH // 16):
                    sl = pl.ds(q * 16, 16)
                    htile[i, sl] = htile[i, sl] * dv
                return _
            lax.fori_loop(0, SUBN, _srow, None)

        def _hop0_sub(r, _):
            nsl = pl.ds(nbase + r * SUBN, SUBN)
            pltpu.sync_copy(xs_hbm.at[c, nsl], htile)
            pltpu.sync_copy(htile, out_hbm.at[c, 0, nsl])
            _scale_sub(r)
            pltpu.sync_copy(htile, g_sh.at[nsl])
            pltpu.sync_copy(htile, a_sh.at[nsl])
            return _
        lax.fori_loop(0, NSUB, _hop0_sub, None)
        plsc.subcore_barrier()

        def _hop(k, _):
            for jj in range(5):
                idx_start(jj, jj)
            idx_wait(0, 0)
            gat_start(0, 0)
            idx_wait(1, 1)
            gat_start(1, 1)

            def _edge_outer(jo, _):
                for u in range(6):
                    j6 = jo * 6 + u
                    m = u
                    p = u % 3

                    @pl.when(j6 + 2 < NCHUNK)
                    def _():
                        idx_wait(j6 + 2, (m + 2) % 6)

                        @pl.when(j6 >= 1)
                        def _():
                            sca_wait((m + 5) % 6, (p + 2) % 3)
                        gat_start((m + 2) % 6, (p + 2) % 3)

                    @pl.when(j6 + 5 < NCHUNK)
                    def _():
                        idx_start(j6 + 5, (m + 5) % 6)
                    gat_wait(m, p)
                    sca_start(m, p)
                return _
            lax.fori_loop(0, NCHUNK // 6, _edge_outer, None)
            sca_wait((NCHUNK - 3) % 6, (NCHUNK - 3) % 3)
            sca_wait((NCHUNK - 2) % 6, (NCHUNK - 2) % 3)
            sca_wait((NCHUNK - 1) % 6, (NCHUNK - 1) % 3)
            plsc.subcore_barrier()

            def _scale_pass(r, _):
                nsl = pl.ds(nbase + r * SUBN, SUBN)
                pltpu.sync_copy(a_sh.at[nsl], htile)
                _scale_sub(r)
                pltpu.sync_copy(htile, out_hbm.at[c, k, nsl])
                _scale_sub(r)
                pltpu.sync_copy(htile, g_sh.at[nsl])
                pltpu.sync_copy(htile, a_sh.at[nsl])
                return _
            lax.fori_loop(0, NSUB, _scale_pass, None)
            plsc.subcore_barrier()
            return _
        lax.fori_loop(1, K_HOPS + 1, _hop, None)

    pl.run_scoped(
        run,
        sem_i=pltpu.SemaphoreType.DMA((6,)),
        sem_g=pltpu.SemaphoreType.DMA((3,)),
        sem_s=pltpu.SemaphoreType.DMA((3,)),
    )


@jax.jit
def _sc_hops(xsplit, rc_p):
    mesh = plsc.VectorSubcoreMesh(
        core_axis_name="c", subcore_axis_name="s",
        num_cores=NC, num_subcores=NS)
    return pl.kernel(
        _sc_body,
        out_type=jax.ShapeDtypeStruct((NC, K_HOPS + 1, NPADN, DH),
                                      jnp.float32),
        mesh=mesh,
        compiler_params=pltpu.CompilerParams(use_tc_tiling_on_sc=False),
        scratch_types=[
            pltpu.VMEM_SHARED((NPADN, DH), jnp.float32),
            pltpu.VMEM_SHARED((NPADN, DH), jnp.float32),
            pltpu.VMEM((6, 2, CHUNK), jnp.int32),
            pltpu.VMEM((NPT, 16), jnp.float32),
            pltpu.VMEM((SUBN, DH), jnp.float32),
            pltpu.VMEM((3, CHUNK, DH), jnp.float32),
        ],
    )(xsplit, rc_p)


def _readout_body(hs_ref, w_ref, b_ref, out_ref):
    hb = hs_ref[...]
    h = jnp.concatenate([hb[0], hb[1]], axis=-1)
    kk, bn, d = h.shape
    logits = jax.lax.dot_general(
        h.reshape(kk * bn, d), w_ref[...],
        (((1,), (0,)), ((), ())),
        preferred_element_type=jnp.float32)
    sig = jax.nn.sigmoid(logits + b_ref[0, 0]).reshape(kk, bn, 1)
    out_ref[...] = jnp.sum(sig * h, axis=0)


BN = 1024


@jax.jit
def _readout(hs, w, b2):
    grid = (NPADN // BN,)
    return pl.pallas_call(
        _readout_body,
        grid=grid,
        in_specs=[
            pl.BlockSpec((NC, K_HOPS + 1, BN, DH), lambda i: (0, 0, i, 0)),
            pl.BlockSpec((D_FEAT, 1), lambda i: (0, 0)),
            pl.BlockSpec((1, 1), lambda i: (0, 0)),
        ],
        out_specs=pl.BlockSpec((BN, D_FEAT), lambda i: (i, 0)),
        out_shape=jax.ShapeDtypeStruct((NPADN, D_FEAT), jnp.float32),
    )(hs, w, b2)


def kernel(x, edge_index, W, b):
    ei = edge_index.astype(jnp.int32)
    row, col = ei[0], ei[1]
    pad = EPT_PAD * NS - N_EDGES
    row_p = jnp.concatenate(
        [row, jnp.zeros((pad,), jnp.int32)]).reshape(NS, NCHUNK, CHUNK)
    col_p = jnp.concatenate(
        [col, jnp.full((pad,), TRASH, jnp.int32)]).reshape(NS, NCHUNK, CHUNK)
    rc_p = jnp.stack([row_p, col_p], axis=2)
    xp = jnp.pad(x, ((0, NPADN - N_NODES), (0, 0)))
    xsplit = xp.reshape(NPADN, NC, DH).transpose(1, 0, 2)
    hs = _sc_hops(xsplit, rc_p)
    return _readout(hs, W, b.reshape(1, 1))[:N_NODES]


# speedup vs baseline: 14.8840x; 1.0450x over previous
import jax
import jax.numpy as jnp
from jax import lax
from jax.experimental import pallas as pl
from jax.experimental.pallas import tpu as pltpu
from jax.experimental.pallas import tpu_sc as plsc

N_NODES = 10000
D_FEAT = 128
N_EDGES = 320000
K_HOPS = 10

NC = 2
NS = 16
DH = D_FEAT // NC
NPADN = 10240
NPT = NPADN // NS
SUBN = 80
NSUB = NPT // SUBN
EPT = N_EDGES // NS
CHUNK = 128
NCHUNK = 160
EPT_PAD = NCHUNK * CHUNK
TRASH = N_NODES


def _sc_body(xs_hbm, rc_hbm, out_hbm, dinv_hbm,
             g_sh, a_sh,
             idxr, dinv_v, dinv2_v, htile, bufr):
    c = lax.axis_index("c")
    s = lax.axis_index("s")
    nbase = s * NPT

    def run(sem_i, sem_g, sem_s, sem_l, sem_t, sem_o):
        def idx_start(j, m):
            pltpu.async_copy(rc_hbm.at[s, j], idxr.at[m], sem_i.at[m])

        def idx_wait(j, m):
            pltpu.make_async_copy(rc_hbm.at[s, j], idxr.at[m],
                                  sem_i.at[m]).wait()

        def gat_start(m, p):
            pltpu.async_copy(g_sh.at[idxr.at[m, 0]], bufr.at[p],
                             sem_g.at[p])

        def gat_wait(m, p):
            pltpu.make_async_copy(g_sh.at[idxr.at[m, 0]], bufr.at[p],
                                  sem_g.at[p]).wait()

        def sca_start(m, p):
            pltpu.async_copy(bufr.at[p], a_sh.at[idxr.at[m, 1]],
                             sem_s.at[p], add=True)

        def sca_wait(m, p):
            pltpu.make_async_copy(bufr.at[p], a_sh.at[idxr.at[m, 1]],
                                  sem_s.at[p]).wait()

        def out_start(k):
            pltpu.async_copy(g_sh.at[pl.ds(nbase, NPT)],
                             out_hbm.at[c, k, pl.ds(nbase, NPT)], sem_o)

        def out_wait(k):
            pltpu.make_async_copy(g_sh.at[pl.ds(nbase, NPT)],
                                  out_hbm.at[c, k, pl.ds(nbase, NPT)],
                                  sem_o).wait()

        def _zero(n, _):
            for q in range(DH // 16):
                htile[0, n, pl.ds(q * 16, 16)] = jnp.zeros((16,), jnp.float32)
            return _
        lax.fori_loop(0, SUBN, _zero, None)

        def _zslice(r, _):
            pltpu.sync_copy(htile.at[0],
                            a_sh.at[pl.ds(nbase + r * SUBN, SUBN)])
            return _
        lax.fori_loop(0, NSUB, _zslice, None)

        def _ones(n, _):
            for q in range(DH // 16):
                bufr[0, n, pl.ds(q * 16, 16)] = jnp.ones((16,), jnp.float32)
            return _
        lax.fori_loop(0, CHUNK, _ones, None)
        plsc.subcore_barrier()

        idx_start(0, 0)
        idx_start(1, 1)

        def _deg_outer(jo, _):
            for u in range(4):
                j = jo * 4 + u
                m = u
                idx_wait(j, m)
                pltpu.async_copy(bufr.at[0], a_sh.at[idxr.at[m, 1]],
                                 sem_s.at[u % 2], add=True)

                @pl.when(j >= 2)
                def _():
                    pltpu.make_async_copy(
                        bufr.at[0], a_sh.at[idxr.at[(m + 2) % 4, 1]],
                        sem_s.at[u % 2]).wait()

                @pl.when(j + 2 < NCHUNK)
                def _():
                    idx_start(j + 2, (m + 2) % 4)
            return _
        lax.fori_loop(0, NCHUNK // 4, _deg_outer, None)
        pltpu.make_async_copy(bufr.at[0], a_sh.at[idxr.at[2, 1]],
                              sem_s.at[0]).wait()
        pltpu.make_async_copy(bufr.at[0], a_sh.at[idxr.at[3, 1]],
                              sem_s.at[1]).wait()
        plsc.subcore_barrier()

        def _dinv_sub(r, _):
            pltpu.sync_copy(a_sh.at[pl.ds(nbase + r * SUBN, SUBN)],
                            htile.at[0])

            def _rsqrt(i, _):
                d = htile[0, i, pl.ds(0, 16)] + 1.0
                ss = 0.5 * (d + 1.0)
                for _it in range(12):
                    ss = 0.5 * (ss + d / ss)
                y = 1.0 / ss
                dinv_v[r * SUBN + i, :] = y
                dinv2_v[r * SUBN + i, :] = y * y
                return _
            lax.fori_loop(0, SUBN, _rsqrt, None)
            return _
        lax.fori_loop(0, NSUB, _dinv_sub, None)
        pltpu.sync_copy(dinv_v, dinv_hbm.at[pl.ds(nbase, NPT)])

        def _scale_pass(coef_v, from_x):
            def load_start(r, b):
                if from_x:
                    pltpu.async_copy(
                        xs_hbm.at[c, pl.ds(nbase + r * SUBN, SUBN)],
                        htile.at[b], sem_l.at[b])
                else:
                    pltpu.async_copy(
                        a_sh.at[pl.ds(nbase + r * SUBN, SUBN)],
                        htile.at[b], sem_l.at[b])

            def load_wait(r, b):
                if from_x:
                    pltpu.make_async_copy(
                        xs_hbm.at[c, pl.ds(nbase + r * SUBN, SUBN)],
                        htile.at[b], sem_l.at[b]).wait()
                else:
                    pltpu.make_async_copy(
                        a_sh.at[pl.ds(nbase + r * SUBN, SUBN)],
                        htile.at[b], sem_l.at[b]).wait()

            def st_start(r, b, dst):
                pltpu.async_copy(htile.at[b],
                                 dst.at[pl.ds(nbase + r * SUBN, SUBN)],
                                 sem_t.at[b])

            def st_wait(r, b, dst):
                pltpu.make_async_copy(htile.at[b],
                                      dst.at[pl.ds(nbase + r * SUBN, SUBN)],
                                      sem_t.at[b]).wait()

            load_start(0, 0)
            for r in range(NSUB):
                b = r % 2
                if r + 1 < NSUB:
                    if r >= 1:
                        st_wait(r - 1, (r - 1) % 2, g_sh)
                        st_wait(r - 1, (r - 1) % 2, a_sh)
                    load_start(r + 1, (r + 1) % 2)
                load_wait(r, b)

                def _srow(i, _):
                    dv = coef_v[r * SUBN + i, :]
                    for q in range(DH // 16):
                        sl = pl.ds(q * 16, 16)
                        htile[b, i, sl] = htile[b, i, sl] * dv
                    return _
                lax.fori_loop(0, SUBN, _srow, None)
                st_start(r, b, g_sh)
                st_start(r, b, a_sh)
            for r in (NSUB - 2, NSUB - 1):
                st_wait(r, r % 2, g_sh)
                st_wait(r, r % 2, a_sh)

        _scale_pass(dinv_v, True)
        out_start(0)
        plsc.subcore_barrier()

        def _hop(k, _):
            idx_start(0, 0)
            idx_start(1, 1)
            idx_start(2, 2)
            idx_wait(0, 0)
            gat_start(0, 0)

            def _edge_outer(jo, _):
                for u in range(4):
                    j4 = jo * 4 + u
                    m = u
                    p = u % 2

                    @pl.when(j4 + 1 < NCHUNK)
                    def _():
                        idx_wait(j4 + 1, (m + 1) % 4)

                        @pl.when(j4 >= 1)
                        def _():
                            sca_wait((m + 3) % 4, (p + 1) % 2)
                        gat_start((m + 1) % 4, (p + 1) % 2)

                    @pl.when(j4 + 3 < NCHUNK)
                    def _():
                        idx_start(j4 + 3, (m + 3) % 4)
                    gat_wait(m, p)
                    sca_start(m, p)
                return _
            lax.fori_loop(0, NCHUNK // 4, _edge_outer, None)
            sca_wait(2, 0)
            sca_wait(3, 1)
            plsc.subcore_barrier()

            out_wait(k - 1)
            _scale_pass(dinv2_v, False)
            out_start(k)
            plsc.subcore_barrier()
            return _
        lax.fori_loop(1, K_HOPS + 1, _hop, None)
        out_wait(K_HOPS)

    pl.run_scoped(
        run,
        sem_i=pltpu.SemaphoreType.DMA((4,)),
        sem_g=pltpu.SemaphoreType.DMA((2,)),
        sem_s=pltpu.SemaphoreType.DMA((2,)),
        sem_l=pltpu.SemaphoreType.DMA((2,)),
        sem_t=pltpu.SemaphoreType.DMA((2,)),
        sem_o=pltpu.SemaphoreType.DMA,
    )


@jax.jit
def _sc_hops(xsplit, rc_p):
    mesh = plsc.VectorSubcoreMesh(
        core_axis_name="c", subcore_axis_name="s",
        num_cores=NC, num_subcores=NS)
    return pl.kernel(
        _sc_body,
        out_type=(
            jax.ShapeDtypeStruct((NC, K_HOPS + 1, NPADN, DH), jnp.float32),
            jax.ShapeDtypeStruct((NPADN, 16), jnp.float32),
        ),
        mesh=mesh,
        compiler_params=pltpu.CompilerParams(use_tc_tiling_on_sc=False),
        scratch_types=[
            pltpu.VMEM_SHARED((NPADN, DH), jnp.float32),
            pltpu.VMEM_SHARED((NPADN, DH), jnp.float32),
            pltpu.VMEM((4, 2, CHUNK), jnp.int32),
            pltpu.VMEM((NPT, 16), jnp.float32),
            pltpu.VMEM((NPT, 16), jnp.float32),
            pltpu.VMEM((2, SUBN, DH), jnp.float32),
            pltpu.VMEM((2, CHUNK, DH), jnp.float32),
        ],
    )(xsplit, rc_p)


def _readout_body(hs_ref, dinv_ref, w_ref, b_ref, out_ref):
    hb = hs_ref[...]
    sinv = (1.0 / dinv_ref[:, 0:1])[None]
    h0 = hb[0] * sinv
    h1 = hb[1] * sinv
    h = jnp.concatenate([h0, h1], axis=-1)
    kk, bn, d = h.shape
    logits = jax.lax.dot_general(
        h.reshape(kk * bn, d), w_ref[...],
        (((1,), (0,)), ((), ())),
        preferred_element_type=jnp.float32)
    sig = jax.nn.sigmoid(logits + b_ref[0, 0]).reshape(kk, bn, 1)
    out_ref[...] = jnp.sum(sig * h, axis=0)


BN = 1024


@jax.jit
def _readout(hs, dinv, w, b2):
    grid = (NPADN // BN,)
    return pl.pallas_call(
        _readout_body,
        grid=grid,
        in_specs=[
            pl.BlockSpec((NC, K_HOPS + 1, BN, DH), lambda i: (0, 0, i, 0)),
            pl.BlockSpec((BN, 16), lambda i: (i, 0)),
            pl.BlockSpec((D_FEAT, 1), lambda i: (0, 0)),
            pl.BlockSpec((1, 1), lambda i: (0, 0)),
        ],
        out_specs=pl.BlockSpec((BN, D_FEAT), lambda i: (i, 0)),
        out_shape=jax.ShapeDtypeStruct((NPADN, D_FEAT), jnp.float32),
    )(hs, dinv, w, b2)


def kernel(x, edge_index, W, b):
    ei = edge_index.astype(jnp.int32)
    row, col = ei[0], ei[1]
    pad = EPT_PAD * NS - N_EDGES
    row_p = jnp.concatenate(
        [row, jnp.zeros((pad,), jnp.int32)]).reshape(NS, NCHUNK, CHUNK)
    col_p = jnp.concatenate(
        [col, jnp.full((pad,), TRASH, jnp.int32)]).reshape(NS, NCHUNK, CHUNK)
    rc_p = jnp.stack([row_p, col_p], axis=2)
    xp = jnp.pad(x, ((0, NPADN - N_NODES), (0, 0)))
    xsplit = xp.reshape(NPADN, NC, DH).transpose(1, 0, 2)
    hs, dinv = _sc_hops(xsplit, rc_p)
    return _readout(hs, dinv, W, b.reshape(1, 1))[:N_NODES]


# speedup vs baseline: 15.3440x; 1.0309x over previous
import jax
import jax.numpy as jnp
from jax import lax
from jax.experimental import pallas as pl
from jax.experimental.pallas import tpu as pltpu
from jax.experimental.pallas import tpu_sc as plsc

N_NODES = 10000
D_FEAT = 128
N_EDGES = 320000
K_HOPS = 10

NC = 2
NS = 16
DH = D_FEAT // NC
NPADN = 10240
NPT = NPADN // NS
SUBN = 80
NSUB = NPT // SUBN
EPT = N_EDGES // NS
CHUNK = 128
NCHUNK = 160
EPT_PAD = NCHUNK * CHUNK
TRASH = N_NODES


def _sc_body(xs_hbm, rc_hbm, out_hbm, dinv_hbm,
             g_sh, a_sh,
             idxr, dinv_v, dinv2_v, htile, bufr):
    c = lax.axis_index("c")
    s = lax.axis_index("s")
    nbase = s * NPT

    def run(sem_i, sem_g, sem_s, sem_l, sem_t, sem_o):
        def idx_start(j, m):
            pltpu.async_copy(rc_hbm.at[s, j], idxr.at[m], sem_i.at[m])

        def idx_wait(j, m):
            pltpu.make_async_copy(rc_hbm.at[s, j], idxr.at[m],
                                  sem_i.at[m]).wait()

        def gat_start(m, p):
            pltpu.async_copy(g_sh.at[idxr.at[m, 0]], bufr.at[p],
                             sem_g.at[p])

        def gat_wait(m, p):
            pltpu.make_async_copy(g_sh.at[idxr.at[m, 0]], bufr.at[p],
                                  sem_g.at[p]).wait()

        def sca_start(m, p):
            pltpu.async_copy(bufr.at[p], a_sh.at[idxr.at[m, 1]],
                             sem_s.at[p], add=True)

        def sca_wait(m, p):
            pltpu.make_async_copy(bufr.at[p], a_sh.at[idxr.at[m, 1]],
                                  sem_s.at[p]).wait()

        def out_start(k):
            pltpu.async_copy(g_sh.at[pl.ds(nbase, NPT)],
                             out_hbm.at[c, k, pl.ds(nbase, NPT)], sem_o)

        def out_wait(k):
            pltpu.make_async_copy(g_sh.at[pl.ds(nbase, NPT)],
                                  out_hbm.at[c, k, pl.ds(nbase, NPT)],
                                  sem_o).wait()

        def _zero(n, _):
            for q in range(DH // 16):
                htile[0, n, pl.ds(q * 16, 16)] = jnp.zeros((16,), jnp.float32)
            return _
        lax.fori_loop(0, SUBN, _zero, None)

        def _zslice(r, _):
            pltpu.sync_copy(htile.at[0],
                            a_sh.at[pl.ds(nbase + r * SUBN, SUBN)])
            return _
        lax.fori_loop(0, NSUB, _zslice, None)

        def _ones(n, _):
            for q in range(DH // 16):
                bufr[0, n, pl.ds(q * 16, 16)] = jnp.ones((16,), jnp.float32)
            return _
        lax.fori_loop(0, CHUNK, _ones, None)
        plsc.subcore_barrier()

        idx_start(0, 0)
        idx_start(1, 1)

        def _deg_outer(jo, _):
            for u in range(4):
                j = jo * 4 + u
                m = u
                idx_wait(j, m)
                pltpu.async_copy(bufr.at[0], a_sh.at[idxr.at[m, 1]],
                                 sem_s.at[u % 2], add=True)

                @pl.when(j >= 2)
                def _():
                    pltpu.make_async_copy(
                        bufr.at[0], a_sh.at[idxr.at[(m + 2) % 4, 1]],
                        sem_s.at[u % 2]).wait()

                @pl.when(j + 2 < NCHUNK)
                def _():
                    idx_start(j + 2, (m + 2) % 4)
            return _
        lax.fori_loop(0, NCHUNK // 4, _deg_outer, None)
        pltpu.make_async_copy(bufr.at[0], a_sh.at[idxr.at[2, 1]],
                              sem_s.at[0]).wait()
        pltpu.make_async_copy(bufr.at[0], a_sh.at[idxr.at[3, 1]],
                              sem_s.at[1]).wait()
        plsc.subcore_barrier()

        def _dinv_sub(r, _):
            pltpu.sync_copy(a_sh.at[pl.ds(nbase + r * SUBN, SUBN)],
                            htile.at[0])

            def _rsqrt(i, _):
                d = htile[0, i, pl.ds(0, 16)] + 1.0
                y = jnp.maximum(1.06066017 - 0.17677670 * d,
                                0.21213203 - 0.00141421 * d)
                y = jnp.maximum(y, 0.04242641 - 1.13137085e-5 * d)
                y = jnp.maximum(y, 0.01060660 - 1.76776695e-7 * d)
                y = jnp.maximum(y, 1.6e-3)
                for _it in range(6):
                    y = y * (1.5 - 0.5 * d * y * y)
                dinv_v[r * SUBN + i, :] = y
                dinv2_v[r * SUBN + i, :] = y * y
                return _
            lax.fori_loop(0, SUBN, _rsqrt, None)
            return _
        lax.fori_loop(0, NSUB, _dinv_sub, None)
        pltpu.sync_copy(dinv_v, dinv_hbm.at[pl.ds(nbase, NPT)])

        def _scale_pass(coef_v, from_x):
            def load_start(r, b):
                if from_x:
                    pltpu.async_copy(
                        xs_hbm.at[c, pl.ds(nbase + r * SUBN, SUBN)],
                        htile.at[b], sem_l.at[b])
                else:
                    pltpu.async_copy(
                        a_sh.at[pl.ds(nbase + r * SUBN, SUBN)],
                        htile.at[b], sem_l.at[b])

            def load_wait(r, b):
                if from_x:
                    pltpu.make_async_copy(
                        xs_hbm.at[c, pl.ds(nbase + r * SUBN, SUBN)],
                        htile.at[b], sem_l.at[b]).wait()
                else:
                    pltpu.make_async_copy(
                        a_sh.at[pl.ds(nbase + r * SUBN, SUBN)],
                        htile.at[b], sem_l.at[b]).wait()

            def st_start(r, b, dst):
                pltpu.async_copy(htile.at[b],
                                 dst.at[pl.ds(nbase + r * SUBN, SUBN)],
                                 sem_t.at[b])

            def st_wait(r, b, dst):
                pltpu.make_async_copy(htile.at[b],
                                      dst.at[pl.ds(nbase + r * SUBN, SUBN)],
                                      sem_t.at[b]).wait()

            load_start(0, 0)
            for r in range(NSUB):
                b = r % 2
                if r + 1 < NSUB:
                    if r >= 1:
                        st_wait(r - 1, (r - 1) % 2, g_sh)
                        st_wait(r - 1, (r - 1) % 2, a_sh)
                    load_start(r + 1, (r + 1) % 2)
                load_wait(r, b)

                def _srow(i, _):
                    dv = coef_v[r * SUBN + i, :]
                    for q in range(DH // 16):
                        sl = pl.ds(q * 16, 16)
                        htile[b, i, sl] = htile[b, i, sl] * dv
                    return _
                lax.fori_loop(0, SUBN, _srow, None)
                st_start(r, b, g_sh)
                st_start(r, b, a_sh)
            for r in (NSUB - 2, NSUB - 1):
                st_wait(r, r % 2, g_sh)
                st_wait(r, r % 2, a_sh)

        _scale_pass(dinv_v, True)
        out_start(0)
        plsc.subcore_barrier()

        def _hop(k, _):
            idx_start(0, 0)
            idx_start(1, 1)
            idx_start(2, 2)
            idx_wait(0, 0)
            gat_start(0, 0)

            def _edge_outer(jo, _):
                for u in range(4):
                    j4 = jo * 4 + u
                    m = u
                    p = u % 2

                    @pl.when(j4 + 1 < NCHUNK)
                    def _():
                        idx_wait(j4 + 1, (m + 1) % 4)

                        @pl.when(j4 >= 1)
                        def _():
                            sca_wait((m + 3) % 4, (p + 1) % 2)
                        gat_start((m + 1) % 4, (p + 1) % 2)

                    @pl.when(j4 + 3 < NCHUNK)
                    def _():
                        idx_start(j4 + 3, (m + 3) % 4)
                    gat_wait(m, p)
                    sca_start(m, p)
                return _
            lax.fori_loop(0, NCHUNK // 4, _edge_outer, None)
            sca_wait(2, 0)
            sca_wait(3, 1)
            plsc.subcore_barrier()

            out_wait(k - 1)
            _scale_pass(dinv2_v, False)
            out_start(k)
            plsc.subcore_barrier()
            return _
        lax.fori_loop(1, K_HOPS + 1, _hop, None)
        out_wait(K_HOPS)

    pl.run_scoped(
        run,
        sem_i=pltpu.SemaphoreType.DMA((4,)),
        sem_g=pltpu.SemaphoreType.DMA((2,)),
        sem_s=pltpu.SemaphoreType.DMA((2,)),
        sem_l=pltpu.SemaphoreType.DMA((2,)),
        sem_t=pltpu.SemaphoreType.DMA((2,)),
        sem_o=pltpu.SemaphoreType.DMA,
    )


@jax.jit
def _sc_hops(xsplit, rc_p):
    mesh = plsc.VectorSubcoreMesh(
        core_axis_name="c", subcore_axis_name="s",
        num_cores=NC, num_subcores=NS)
    return pl.kernel(
        _sc_body,
        out_type=(
            jax.ShapeDtypeStruct((NC, K_HOPS + 1, NPADN, DH), jnp.float32),
            jax.ShapeDtypeStruct((NPADN, 16), jnp.float32),
        ),
        mesh=mesh,
        compiler_params=pltpu.CompilerParams(use_tc_tiling_on_sc=False),
        scratch_types=[
            pltpu.VMEM_SHARED((NPADN, DH), jnp.float32),
            pltpu.VMEM_SHARED((NPADN, DH), jnp.float32),
            pltpu.VMEM((4, 2, CHUNK), jnp.int32),
            pltpu.VMEM((NPT, 16), jnp.float32),
            pltpu.VMEM((NPT, 16), jnp.float32),
            pltpu.VMEM((2, SUBN, DH), jnp.float32),
            pltpu.VMEM((2, CHUNK, DH), jnp.float32),
        ],
    )(xsplit, rc_p)


def _readout_body(hs_ref, dinv_ref, w_ref, b_ref, out_ref):
    hb = hs_ref[...]
    sinv = (1.0 / dinv_ref[:, 0:1])[None]
    h0 = hb[0] * sinv
    h1 = hb[1] * sinv
    h = jnp.concatenate([h0, h1], axis=-1)
    kk, bn, d = h.shape
    logits = jax.lax.dot_general(
        h.reshape(kk * bn, d), w_ref[...],
        (((1,), (0,)), ((), ())),
        preferred_element_type=jnp.float32)
    sig = jax.nn.sigmoid(logits + b_ref[0, 0]).reshape(kk, bn, 1)
    out_ref[...] = jnp.sum(sig * h, axis=0)


BN = 1024


@jax.jit
def _readout(hs, dinv, w, b2):
    grid = (NPADN // BN,)
    return pl.pallas_call(
        _readout_body,
        grid=grid,
        in_specs=[
            pl.BlockSpec((NC, K_HOPS + 1, BN, DH), lambda i: (0, 0, i, 0)),
            pl.BlockSpec((BN, 16), lambda i: (i, 0)),
            pl.BlockSpec((D_FEAT, 1), lambda i: (0, 0)),
            pl.BlockSpec((1, 1), lambda i: (0, 0)),
        ],
        out_specs=pl.BlockSpec((BN, D_FEAT), lambda i: (i, 0)),
        out_shape=jax.ShapeDtypeStruct((NPADN, D_FEAT), jnp.float32),
    )(hs, dinv, w, b2)


def kernel(x, edge_index, W, b):
    ei = edge_index.astype(jnp.int32)
    row, col = ei[0], ei[1]
    pad = EPT_PAD * NS - N_EDGES
    row_p = jnp.concatenate(
        [row, jnp.zeros((pad,), jnp.int32)]).reshape(NS, NCHUNK, CHUNK)
    col_p = jnp.concatenate(
        [col, jnp.full((pad,), TRASH, jnp.int32)]).reshape(NS, NCHUNK, CHUNK)
    rc_p = jnp.stack([row_p, col_p], axis=2)
    xp = jnp.pad(x, ((0, NPADN - N_NODES), (0, 0)))
    xsplit = xp.reshape(NPADN, NC, DH).transpose(1, 0, 2)
    hs, dinv = _sc_hops(xsplit, rc_p)
    return _readout(hs, dinv, W, b.reshape(1, 1))[:N_NODES]
